# baseline (device time: 72082 ns/iter reference)
import os

import jax
import jax.numpy as jnp
from jax import lax
from jax.experimental import pallas as pl
from jax.experimental.pallas import tpu as pltpu

N_DEV = 16

_VARIANT = os.environ.get("KVARIANT", "full")
_DO_COMPUTE = _VARIANT in ("full", "compute_only")
_DO_COMM = _VARIANT in ("full", "comm_only")


def kernel(x, w_mat, scale_x, scale_w):
    m_per, k = x.shape
    _, n = w_mat.shape
    n_per = n // N_DEV

    def body(x_ref, w_ref, sx_ref, sw_ref, out_ref,
             wbuf, y_ref, rbuf, load_sems, send_sems, recv_sems):
        me = lax.axis_index("i")

        if _DO_COMM:
            barrier = pltpu.get_barrier_semaphore()
            for d in range(1, N_DEV):
                p = (me + d) % N_DEV
                pl.semaphore_signal(
                    barrier, inc=1, device_id=(p,),
                    device_id_type=pl.DeviceIdType.MESH,
                )
            pl.semaphore_wait(barrier, N_DEV - 1)

        def start_load(slot, j):
            cp = pltpu.make_async_copy(
                w_ref.at[:, pl.ds(j * n_per, n_per)],
                wbuf.at[slot],
                load_sems.at[slot],
            )
            cp.start()
            return cp

        s = sx_ref[0] * sw_ref[0]
        xb = x_ref[...].astype(jnp.bfloat16)

        sched = [8, 9, 7, 10, 6, 11, 5, 12, 4, 13, 3, 14, 2, 15, 1, 0]
        arrival = [(N_DEV - d) % N_DEV for d in sched if d != 0]

        NBUF = 3
        loads = [None] * NBUF
        if _DO_COMPUTE:
            for t in range(NBUF - 1):
                loads[t] = start_load(t, (me + sched[t]) % N_DEV)

        sends = []
        for step, d in enumerate(sched):
            j = (me + d) % N_DEV
            slot = step % NBUF
            if _DO_COMPUTE:
                if step + NBUF - 1 < N_DEV:
                    nxt = (step + NBUF - 1) % NBUF
                    loads[nxt] = start_load(
                        nxt, (me + sched[step + NBUF - 1]) % N_DEV
                    )
                loads[slot].wait()
                yj = jnp.dot(
                    xb,
                    wbuf[slot].astype(jnp.bfloat16),
                    preferred_element_type=jnp.float32,
                )
                yj = yj * s
                yj = yj * (1.0 / (1.0 + jnp.exp(-jnp.clip(yj, -60.0, 60.0))))
            else:
                yj = jnp.zeros((m_per, n_per), jnp.float32)

            if d == 0:
                out_ref[pl.ds(me * m_per, m_per), :] = yj
            elif _DO_COMM:
                y_ref[pl.ds(j, 1), :, :] = yj.astype(jnp.bfloat16)[None]
                rdma = pltpu.make_async_remote_copy(
                    src_ref=y_ref.at[j],
                    dst_ref=rbuf.at[me],
                    send_sem=send_sems.at[d],
                    recv_sem=recv_sems.at[me],
                    device_id=(j,),
                    device_id_type=pl.DeviceIdType.MESH,
                )
                rdma.start()
                sends.append(rdma)
            else:
                y_ref[pl.ds(j, 1), :, :] = yj.astype(jnp.bfloat16)[None]

        for d in arrival:
            p = (me + d) % N_DEV
            if _DO_COMM:
                recv = pltpu.make_async_remote_copy(
                    src_ref=y_ref.at[p],
                    dst_ref=rbuf.at[p],
                    send_sem=send_sems.at[d],
                    recv_sem=recv_sems.at[p],
                    device_id=(p,),
                    device_id_type=pl.DeviceIdType.MESH,
                )
                recv.wait_recv()
            out_ref[pl.ds(p * m_per, m_per), :] = rbuf[p].astype(jnp.float32)

        for rdma in sends:
            rdma.wait_send()

    return pl.pallas_call(
        body,
        out_shape=jax.ShapeDtypeStruct((N_DEV * m_per, n_per), jnp.float32),
        in_specs=[
            pl.BlockSpec(memory_space=pltpu.VMEM),
            pl.BlockSpec(memory_space=pl.ANY),
            pl.BlockSpec(memory_space=pltpu.SMEM),
            pl.BlockSpec(memory_space=pltpu.SMEM),
        ],
        out_specs=pl.BlockSpec(memory_space=pltpu.VMEM),
        scratch_shapes=[
            pltpu.VMEM((3, k, n_per), w_mat.dtype),
            pltpu.VMEM((N_DEV, m_per, n_per), jnp.bfloat16),
            pltpu.VMEM((N_DEV, m_per, n_per), jnp.bfloat16),
            pltpu.SemaphoreType.DMA((3,)),
            pltpu.SemaphoreType.DMA((N_DEV,)),
            pltpu.SemaphoreType.DMA((N_DEV,)),
        ],
        compiler_params=pltpu.CompilerParams(
            collective_id=0 if _DO_COMM else None,
            vmem_limit_bytes=100 * 1024 * 1024,
        ),
    )(x, w_mat, scale_x, scale_w)


# device time: 70212 ns/iter; 1.0266x vs baseline; 1.0266x over previous
import os

import jax
import jax.numpy as jnp
from jax import lax
from jax.experimental import pallas as pl
from jax.experimental.pallas import tpu as pltpu

N_DEV = 16

_VARIANT = os.environ.get("KVARIANT", "full")
_DO_LOADS = _VARIANT in ("full", "compute_only", "comm_loads")
_DO_MATMUL = _VARIANT in ("full", "compute_only")
_DO_COMM = _VARIANT in ("full", "comm_only", "comm_loads")


def kernel(x, w_mat, scale_x, scale_w):
    m_per, k = x.shape
    _, n = w_mat.shape
    n_per = n // N_DEV

    def body(x_ref, w_ref, sx_ref, sw_ref, out_ref,
             wbuf, y_ref, rbuf, load_sems, send_sems, recv_sems):
        me = lax.axis_index("i")

        if _DO_COMM:
            barrier = pltpu.get_barrier_semaphore()
            for d in range(1, N_DEV):
                p = (me + d) % N_DEV
                pl.semaphore_signal(
                    barrier, inc=1, device_id=(p,),
                    device_id_type=pl.DeviceIdType.MESH,
                )
            pl.semaphore_wait(barrier, N_DEV - 1)

        def start_load(slot, j):
            cp = pltpu.make_async_copy(
                w_ref.at[:, pl.ds(j * n_per, n_per)],
                wbuf.at[slot],
                load_sems.at[slot],
            )
            cp.start()
            return cp

        s = sx_ref[0] * sw_ref[0]
        xb = x_ref[...].astype(jnp.bfloat16)

        sched = [8, 9, 7, 10, 6, 11, 5, 12, 4, 13, 3, 14, 2, 15, 1, 0]
        arrival = [(N_DEV - d) % N_DEV for d in sched if d != 0]

        NBUF = 3
        loads = [None] * NBUF
        if _DO_LOADS:
            for t in range(NBUF - 1):
                loads[t] = start_load(t, (me + sched[t]) % N_DEV)

        sends = []
        for step, d in enumerate(sched):
            j = (me + d) % N_DEV
            slot = step % NBUF
            if _DO_LOADS:
                if step + NBUF - 1 < N_DEV:
                    nxt = (step + NBUF - 1) % NBUF
                    loads[nxt] = start_load(
                        nxt, (me + sched[step + NBUF - 1]) % N_DEV
                    )
                loads[slot].wait()
            if _DO_MATMUL:
                yj = jnp.dot(
                    xb,
                    wbuf[slot].astype(jnp.bfloat16),
                    preferred_element_type=jnp.float32,
                )
                yj = yj * s
                yj = yj * (1.0 / (1.0 + jnp.exp(-jnp.clip(yj, -60.0, 60.0))))
            else:
                yj = jnp.zeros((m_per, n_per), jnp.float32)

            if d == 0:
                out_ref[pl.ds(me * m_per, m_per), :] = yj
            elif _DO_COMM:
                y_ref[pl.ds(j, 1), :, :] = yj.astype(jnp.bfloat16)[None]
                rdma = pltpu.make_async_remote_copy(
                    src_ref=y_ref.at[j],
                    dst_ref=rbuf.at[me],
                    send_sem=send_sems.at[d],
                    recv_sem=recv_sems.at[me],
                    device_id=(j,),
                    device_id_type=pl.DeviceIdType.MESH,
                )
                rdma.start()
                sends.append(rdma)
            else:
                y_ref[pl.ds(j, 1), :, :] = yj.astype(jnp.bfloat16)[None]

        for d in arrival:
            p = (me + d) % N_DEV
            if _DO_COMM:
                recv = pltpu.make_async_remote_copy(
                    src_ref=y_ref.at[p],
                    dst_ref=rbuf.at[p],
                    send_sem=send_sems.at[d],
                    recv_sem=recv_sems.at[p],
                    device_id=(p,),
                    device_id_type=pl.DeviceIdType.MESH,
                )
                recv.wait_recv()
            out_ref[pl.ds(p * m_per, m_per), :] = rbuf[p].astype(jnp.float32)

        for rdma in sends:
            rdma.wait_send()

    return pl.pallas_call(
        body,
        out_shape=jax.ShapeDtypeStruct((N_DEV * m_per, n_per), jnp.float32),
        in_specs=[
            pl.BlockSpec(memory_space=pltpu.VMEM),
            pl.BlockSpec(memory_space=pl.ANY),
            pl.BlockSpec(memory_space=pltpu.SMEM),
            pl.BlockSpec(memory_space=pltpu.SMEM),
        ],
        out_specs=pl.BlockSpec(memory_space=pltpu.VMEM),
        scratch_shapes=[
            pltpu.VMEM((3, k, n_per), w_mat.dtype),
            pltpu.VMEM((N_DEV, m_per, n_per), jnp.bfloat16),
            pltpu.VMEM((N_DEV, m_per, n_per), jnp.bfloat16),
            pltpu.SemaphoreType.DMA((3,)),
            pltpu.SemaphoreType.DMA((N_DEV,)),
            pltpu.SemaphoreType.DMA((N_DEV,)),
        ],
        compiler_params=pltpu.CompilerParams(
            collective_id=0 if _DO_COMM else None,
            vmem_limit_bytes=100 * 1024 * 1024,
        ),
    )(x, w_mat, scale_x, scale_w)
